# Initial kernel scaffold; baseline (speedup 1.0000x reference)
import jax
import jax.numpy as jnp
from jax.experimental import pallas as pl


def kernel(hidden_states, W, scale):
    raise NotImplementedError("write your pallas kernel here")


# speedup vs baseline: 2.4705x; 2.4705x over previous
import functools

import jax
import jax.numpy as jnp
from jax import lax
from jax.experimental import pallas as pl
from jax.experimental.pallas import tpu as pltpu
from jax.experimental.pallas import tpu_sc as plsc

_H = 768
_E = 64
_TOKENS = 32768
_EPS = 1e-6
_ROOT = _H ** -0.5

_NC, _NS, _L = 2, 16, 16
_NW = _NC * _NS
_TPW = _TOKENS // _NW
_G = _TPW // _L


def _logits_body(x_ref, w_ref, s_ref, lt_ref):
    x = x_ref[...]
    ms = jnp.mean(x * x, axis=1, keepdims=True)
    n = x * jax.lax.rsqrt(ms + _EPS)
    n = n * jnp.float32(_ROOT)
    n = n * s_ref[...]
    nb = n.astype(jnp.bfloat16)
    wb = w_ref[...].astype(jnp.bfloat16)
    lt_ref[...] = jax.lax.dot_general(
        wb, nb, (((1,), (1,)), ((), ())),
        preferred_element_type=jnp.float32)


_sc_mesh = plsc.VectorSubcoreMesh(core_axis_name="c", subcore_axis_name="s")


@functools.partial(
    pl.kernel,
    mesh=_sc_mesh,
    out_type=[jax.ShapeDtypeStruct((2, _TOKENS), jnp.float32),
              jax.ShapeDtypeStruct((2, _TOKENS), jnp.int32)],
    scratch_types=[pltpu.VMEM((_E, _TPW), jnp.float32),
                   pltpu.VMEM((2, _TPW), jnp.float32),
                   pltpu.VMEM((2, _TPW), jnp.int32)],
)
def _sc_topk(lt_hbm, w_hbm, i_hbm, lt_v, w_v, i_v):
    wid = lax.axis_index("s") * _NC + lax.axis_index("c")
    base = wid * _TPW
    pltpu.sync_copy(lt_hbm.at[:, pl.ds(base, _TPW)], lt_v)

    def per_group(g, _):
        col = g * _L

        def scan_e(e, carry):
            m1, i1, m2, i2 = carry
            v = lt_v[e, pl.ds(col, _L)]
            es = jnp.full((_L,), e, jnp.int32)
            gt1 = v > m1
            gt2 = v > m2
            m2n = jnp.where(gt1, m1, jnp.where(gt2, v, m2))
            i2n = jnp.where(gt1, i1, jnp.where(gt2, es, i2))
            m1n = jnp.where(gt1, v, m1)
            i1n = jnp.where(gt1, es, i1)
            return m1n, i1n, m2n, i2n

        m1 = lt_v[0, pl.ds(col, _L)]
        i1 = jnp.zeros((_L,), jnp.int32)
        m2 = jnp.full((_L,), -jnp.inf, jnp.float32)
        i2 = jnp.zeros((_L,), jnp.int32)
        m1, i1, m2, i2 = lax.fori_loop(1, _E, scan_e, (m1, i1, m2, i2))
        ex = jnp.exp(m2 - m1)
        w2 = ex / (1.0 + ex)
        w1 = 1.0 - w2
        w_v[0, pl.ds(col, _L)] = w1
        w_v[1, pl.ds(col, _L)] = w2
        i_v[0, pl.ds(col, _L)] = i1
        i_v[1, pl.ds(col, _L)] = i2
        return 0

    lax.fori_loop(0, _G, per_group, 0)
    pltpu.sync_copy(w_v, w_hbm.at[:, pl.ds(base, _TPW)])
    pltpu.sync_copy(i_v, i_hbm.at[:, pl.ds(base, _TPW)])


def kernel(hidden_states, W, scale):
    Tb = 1024
    lt = pl.pallas_call(
        _logits_body,
        grid=(_TOKENS // Tb,),
        in_specs=[
            pl.BlockSpec((Tb, _H), lambda i: (i, 0)),
            pl.BlockSpec((_E, _H), lambda i: (0, 0)),
            pl.BlockSpec((1, _H), lambda i: (0, 0)),
        ],
        out_specs=pl.BlockSpec((_E, Tb), lambda i: (0, i)),
        out_shape=jax.ShapeDtypeStruct((_E, _TOKENS), jnp.float32),
        compiler_params=pltpu.CompilerParams(
            dimension_semantics=("arbitrary",)),
    )(hidden_states, W, scale.reshape(1, _H))
    w2d, i2d = _sc_topk(lt)
    return (w2d.T, i2d.T)


# speedup vs baseline: 2.6663x; 1.0792x over previous
import functools

import jax
import jax.numpy as jnp
from jax import lax
from jax.experimental import pallas as pl
from jax.experimental.pallas import tpu as pltpu
from jax.experimental.pallas import tpu_sc as plsc

_H = 768
_E = 64
_TOKENS = 32768
_EPS = 1e-6
_ROOT = _H ** -0.5

_NC, _NS, _L = 2, 16, 16
_NW = _NC * _NS
_C = 4
_CT = _TOKENS // _C
_TPW = _CT // _NW
_G = _TPW // _L
_GU = 4


def _logits_body(x_ref, w_ref, s_ref, lt_ref):
    x = x_ref[...]
    ms = jnp.mean(x * x, axis=1, keepdims=True)
    n = x * jax.lax.rsqrt(ms + _EPS)
    n = n * jnp.float32(_ROOT)
    n = n * s_ref[...]
    nb = n.astype(jnp.bfloat16)
    wb = w_ref[...].astype(jnp.bfloat16)
    lt_ref[...] = jax.lax.dot_general(
        wb, nb, (((1,), (1,)), ((), ())),
        preferred_element_type=jnp.float32)


_sc_mesh = plsc.VectorSubcoreMesh(core_axis_name="c", subcore_axis_name="s")


@functools.partial(
    pl.kernel,
    mesh=_sc_mesh,
    out_type=[jax.ShapeDtypeStruct((2, _CT), jnp.float32),
              jax.ShapeDtypeStruct((2, _CT), jnp.int32)],
    scratch_types=[pltpu.VMEM((_E, _TPW), jnp.float32),
                   pltpu.VMEM((2, _TPW), jnp.float32),
                   pltpu.VMEM((2, _TPW), jnp.int32)],
)
def _sc_topk(lt_hbm, w_hbm, i_hbm, lt_v, w_v, i_v):
    wid = lax.axis_index("s") * _NC + lax.axis_index("c")
    base = wid * _TPW
    pltpu.sync_copy(lt_hbm.at[:, pl.ds(base, _TPW)], lt_v)

    def per_block(b, _):
        col0 = b * (_GU * _L)

        def scan_e(e, carry):
            es = jnp.full((_L,), e, jnp.int32)
            out = []
            for u in range(_GU):
                m1, i1, m2, i2 = carry[u]
                v = lt_v[e, pl.ds(col0 + u * _L, _L)]
                gt1 = v > m1
                gt2 = v > m2
                m2n = jnp.where(gt1, m1, jnp.where(gt2, v, m2))
                i2n = jnp.where(gt1, i1, jnp.where(gt2, es, i2))
                m1n = jnp.where(gt1, v, m1)
                i1n = jnp.where(gt1, es, i1)
                out.append((m1n, i1n, m2n, i2n))
            return tuple(out)

        zi = jnp.zeros((_L,), jnp.int32)
        ninf = jnp.full((_L,), -jnp.inf, jnp.float32)
        init = tuple(
            (lt_v[0, pl.ds(col0 + u * _L, _L)], zi, ninf, zi)
            for u in range(_GU))
        res = lax.fori_loop(1, _E, scan_e, init)
        for u in range(_GU):
            m1, i1, m2, i2 = res[u]
            ex = jnp.exp(m2 - m1)
            w2 = ex / (1.0 + ex)
            w1 = 1.0 - w2
            col = col0 + u * _L
            w_v[0, pl.ds(col, _L)] = w1
            w_v[1, pl.ds(col, _L)] = w2
            i_v[0, pl.ds(col, _L)] = i1
            i_v[1, pl.ds(col, _L)] = i2
        return 0

    lax.fori_loop(0, _G // _GU, per_block, 0)
    pltpu.sync_copy(w_v, w_hbm.at[:, pl.ds(base, _TPW)])
    pltpu.sync_copy(i_v, i_hbm.at[:, pl.ds(base, _TPW)])


def kernel(hidden_states, W, scale):
    Tb = 1024
    scale2d = scale.reshape(1, _H)
    ws, idxs = [], []
    for c in range(_C):
        lt_c = pl.pallas_call(
            _logits_body,
            grid=(_CT // Tb,),
            in_specs=[
                pl.BlockSpec((Tb, _H),
                             functools.partial(
                                 lambda i, c: (c * (_CT // Tb) + i, 0), c=c)),
                pl.BlockSpec((_E, _H), lambda i: (0, 0)),
                pl.BlockSpec((1, _H), lambda i: (0, 0)),
            ],
            out_specs=pl.BlockSpec((_E, Tb), lambda i: (0, i)),
            out_shape=jax.ShapeDtypeStruct((_E, _CT), jnp.float32),
            compiler_params=pltpu.CompilerParams(
                dimension_semantics=("arbitrary",)),
        )(hidden_states, W, scale2d)
        w_c, i_c = _sc_topk(lt_c)
        ws.append(w_c)
        idxs.append(i_c)
    w2d = jnp.concatenate(ws, axis=1)
    i2d = jnp.concatenate(idxs, axis=1)
    return (w2d.T, i2d.T)


# speedup vs baseline: 2.9906x; 1.1216x over previous
import functools

import jax
import jax.numpy as jnp
from jax import lax
from jax.experimental import pallas as pl
from jax.experimental.pallas import tpu as pltpu
from jax.experimental.pallas import tpu_sc as plsc

_H = 768
_E = 64
_TOKENS = 32768
_EPS = 1e-6
_ROOT = _H ** -0.5

_NC, _NS, _L = 2, 16, 16
_NW = _NC * _NS
_TPW = _TOKENS // _NW
_NSLAB = 4
_TPS = _TPW // _NSLAB
_GU = 4
_UE = 3


def _logits_body(x_ref, w_ref, s_ref, lt_ref):
    x = x_ref[...]
    ms = jnp.mean(x * x, axis=1, keepdims=True)
    n = x * jax.lax.rsqrt(ms + _EPS)
    n = n * jnp.float32(_ROOT)
    n = n * s_ref[...]
    nb = n.astype(jnp.bfloat16)
    wb = w_ref[...].astype(jnp.bfloat16)
    lt_ref[...] = jax.lax.dot_general(
        wb, nb, (((1,), (1,)), ((), ())),
        preferred_element_type=jnp.float32)


_sc_mesh = plsc.VectorSubcoreMesh(core_axis_name="c", subcore_axis_name="s")


@functools.partial(
    pl.kernel,
    mesh=_sc_mesh,
    out_type=[jax.ShapeDtypeStruct((2, _TOKENS), jnp.float32),
              jax.ShapeDtypeStruct((2, _TOKENS), jnp.int32)],
    scratch_types=[pltpu.VMEM((_E, _TPW), jnp.float32),
                   pltpu.VMEM((2, _TPW), jnp.float32),
                   pltpu.VMEM((2, _TPW), jnp.int32),
                   pltpu.SemaphoreType.DMA((_NSLAB,))],
)
def _sc_topk(lt_hbm, w_hbm, i_hbm, lt_v, w_v, i_v, sems):
    wid = lax.axis_index("s") * _NC + lax.axis_index("c")
    base = wid * _TPW

    copies = [
        pltpu.async_copy(
            lt_hbm.at[:, pl.ds(base + s * _TPS, _TPS)],
            lt_v.at[:, pl.ds(s * _TPS, _TPS)],
            sems.at[s])
        for s in range(_NSLAB)
    ]

    for s in range(_NSLAB):
        copies[s].wait()
        for b in range(_TPS // (_GU * _L)):
            col0 = s * _TPS + b * (_GU * _L)
            cols = [col0 + u * _L for u in range(_GU)]

            def scan_e(it, carry, cols=cols):
                out = carry
                for k in range(_UE):
                    e = 1 + it * _UE + k
                    es = jnp.full((_L,), e, jnp.int32)
                    nxt = []
                    for u in range(_GU):
                        m1, i1, m2, i2 = out[u]
                        v = lt_v[e, pl.ds(cols[u], _L)]
                        gt1 = v > m1
                        gt2 = v > m2
                        m2n = jnp.where(gt1, m1, jnp.where(gt2, v, m2))
                        i2n = jnp.where(gt1, i1, jnp.where(gt2, es, i2))
                        m1n = jnp.where(gt1, v, m1)
                        i1n = jnp.where(gt1, es, i1)
                        nxt.append((m1n, i1n, m2n, i2n))
                    out = tuple(nxt)
                return out

            zi = jnp.zeros((_L,), jnp.int32)
            ninf = jnp.full((_L,), -jnp.inf, jnp.float32)
            init = tuple(
                (lt_v[0, pl.ds(cols[u], _L)], zi, ninf, zi)
                for u in range(_GU))
            res = lax.fori_loop(0, (_E - 1) // _UE, scan_e, init)
            for u in range(_GU):
                m1, i1, m2, i2 = res[u]
                ex = jnp.exp(m2 - m1)
                w2 = ex / (1.0 + ex)
                w1 = 1.0 - w2
                w_v[0, pl.ds(cols[u], _L)] = w1
                w_v[1, pl.ds(cols[u], _L)] = w2
                i_v[0, pl.ds(cols[u], _L)] = i1
                i_v[1, pl.ds(cols[u], _L)] = i2

    pltpu.sync_copy(w_v, w_hbm.at[:, pl.ds(base, _TPW)])
    pltpu.sync_copy(i_v, i_hbm.at[:, pl.ds(base, _TPW)])


def kernel(hidden_states, W, scale):
    Tb = 2048
    lt = pl.pallas_call(
        _logits_body,
        grid=(_TOKENS // Tb,),
        in_specs=[
            pl.BlockSpec((Tb, _H), lambda i: (i, 0)),
            pl.BlockSpec((_E, _H), lambda i: (0, 0)),
            pl.BlockSpec((1, _H), lambda i: (0, 0)),
        ],
        out_specs=pl.BlockSpec((_E, Tb), lambda i: (0, i)),
        out_shape=jax.ShapeDtypeStruct((_E, _TOKENS), jnp.float32),
        compiler_params=pltpu.CompilerParams(
            dimension_semantics=("arbitrary",)),
    )(hidden_states, W, scale.reshape(1, _H))
    w2d, i2d = _sc_topk(lt)
    return (w2d.T, i2d.T)
